# baseline (speedup 1.0000x reference)
import functools

import jax
import jax.numpy as jnp
from jax import lax
from jax.experimental import pallas as pl
from jax.experimental.pallas import tpu as pltpu
from jax.experimental.pallas import tpu_sc as plsc

D_EMB = 1000
ROWS = 4096
COLS = 20
N = ROWS * COLS
NC, NS, L = 2, 16, 16
NW = NC * NS
RPW = N // NW
R = 64
NIT = RPW // R

_mesh = plsc.VectorSubcoreMesh(core_axis_name="c", subcore_axis_name="s")


@functools.partial(
    pl.kernel,
    mesh=_mesh,
    out_type=jax.ShapeDtypeStruct((N * D_EMB,), jnp.float32),
    scratch_types=[
        pltpu.VMEM((RPW,), jnp.int32),
        pltpu.VMEM((R * D_EMB,), jnp.float32),
        pltpu.VMEM((R * D_EMB,), jnp.float32),
        pltpu.SemaphoreType.DMA,
        pltpu.SemaphoreType.DMA,
    ],
    compiler_params=pltpu.CompilerParams(needs_layout_passes=False),
)
def _sc_onehot(x_hbm, out_hbm, xall, buf0, buf1, sem0, sem1):
    wid = lax.axis_index("s") * NC + lax.axis_index("c")
    gbase = wid * RPW
    pltpu.sync_copy(x_hbm.at[pl.ds(gbase, RPW)], xall)

    zeros16 = jnp.zeros((L,), jnp.float32)
    fives16 = jnp.full((L,), 5.0, jnp.float32)
    lane = lax.iota(jnp.int32, L)

    def zbody(k, carry):
        for u in range(16):
            off = (k * 16 + u) * L
            buf0[pl.ds(off, L)] = zeros16
            buf1[pl.ds(off, L)] = zeros16
        return carry

    lax.fori_loop(0, R * D_EMB // (16 * L), zbody, 0)

    def step(t, buf, sem):
        base = gbase + t * R

        @pl.when(t >= 2)
        def _():
            pltpu.make_async_copy(
                buf, out_hbm.at[pl.ds((base - 2 * R) * D_EMB, R * D_EMB)], sem
            ).wait()
            for j in range(R // L):
                xv = xall[pl.ds((t - 2) * R + j * L, L)]
                offs = (lane + j * L) * D_EMB + xv
                plsc.store_scatter(buf, [offs], zeros16)

        for j in range(R // L):
            xv = xall[pl.ds(t * R + j * L, L)]
            offs = (lane + j * L) * D_EMB + xv
            plsc.store_scatter(buf, [offs], fives16)
        pltpu.async_copy(buf, out_hbm.at[pl.ds(base * D_EMB, R * D_EMB)], sem)

    def lbody(k, carry):
        step(2 * k, buf0, sem0)
        step(2 * k + 1, buf1, sem1)
        return carry

    lax.fori_loop(0, NIT // 2, lbody, 0)
    pltpu.make_async_copy(
        buf0, out_hbm.at[pl.ds(gbase * D_EMB, R * D_EMB)], sem0
    ).wait()
    pltpu.make_async_copy(
        buf1, out_hbm.at[pl.ds(gbase * D_EMB, R * D_EMB)], sem1
    ).wait()


def kernel(x):
    flat = _sc_onehot(x.reshape(N))
    return flat.reshape(ROWS, COLS, D_EMB)


# speedup vs baseline: 6.9971x; 6.9971x over previous
import jax
import jax.numpy as jnp
from jax.experimental import pallas as pl
from jax.experimental.pallas import tpu as pltpu

D_EMB = 1000
ROWS = 4096
COLS = 20
CBLK = 256


def _onehot_block(xt_ref, o_ref):
    xb = xt_ref[...]
    iota = jax.lax.broadcasted_iota(jnp.int32, (COLS, D_EMB, CBLK), 1)
    o_ref[...] = jnp.where(xb[:, None, :] == iota, 5.0, 0.0).astype(jnp.float32)


def kernel(x):
    xt = x.T
    out_t = pl.pallas_call(
        _onehot_block,
        grid=(ROWS // CBLK,),
        in_specs=[pl.BlockSpec((COLS, CBLK), lambda i: (0, i))],
        out_specs=pl.BlockSpec((COLS, D_EMB, CBLK), lambda i: (0, 0, i)),
        out_shape=jax.ShapeDtypeStruct((COLS, D_EMB, ROWS), jnp.float32),
        compiler_params=pltpu.CompilerParams(
            dimension_semantics=("parallel",)),
    )(xt)
    return out_t.transpose(2, 0, 1)


# speedup vs baseline: 7.0805x; 1.0119x over previous
import jax
import jax.numpy as jnp
from jax.experimental import pallas as pl
from jax.experimental.pallas import tpu as pltpu

D_EMB = 1000
ROWS = 4096
COLS = 20
CBLK = 128


def _onehot_block(xt_ref, o_ref):
    xb = xt_ref[...]
    iota = jax.lax.broadcasted_iota(jnp.int32, (COLS, D_EMB, CBLK), 1)
    o_ref[...] = jnp.where(xb[:, None, :] == iota, 5.0, 0.0).astype(jnp.float32)


def kernel(x):
    xt = x.T
    out_t = pl.pallas_call(
        _onehot_block,
        grid=(ROWS // CBLK,),
        in_specs=[pl.BlockSpec((COLS, CBLK), lambda i: (0, i))],
        out_specs=pl.BlockSpec((COLS, D_EMB, CBLK), lambda i: (0, 0, i)),
        out_shape=jax.ShapeDtypeStruct((COLS, D_EMB, ROWS), jnp.float32),
        compiler_params=pltpu.CompilerParams(
            dimension_semantics=("parallel",)),
    )(xt)
    return out_t.transpose(2, 0, 1)
